# baseline (speedup 1.0000x reference)
import functools

import jax
import jax.numpy as jnp
from jax import lax
from jax.experimental import pallas as pl
from jax.experimental.pallas import tpu as pltpu
from jax.experimental.pallas import tpu_sc as plsc

_NUM_SUBJECTS = 100000
_EMBED_DIM = 64
_FEATURE_DIM = 128
_BATCH = 4096
_SEQ_LEN = 50

_BB = 256


@functools.cache
def _build_sc_gather(batch, embed_dim):
    info = plsc.get_sparse_core_info()
    nw = info.num_cores * info.num_subcores
    bpw = batch // nw
    mesh = plsc.VectorSubcoreMesh(core_axis_name="c", subcore_axis_name="s")

    k = 16

    @functools.partial(
        pl.kernel,
        mesh=mesh,
        out_type=jax.ShapeDtypeStruct((batch, embed_dim), jnp.float32),
        scratch_types=[
            pltpu.VMEM((bpw,), jnp.int32),
            pltpu.VMEM((bpw, embed_dim), jnp.float32),
            pltpu.SemaphoreType.DMA,
        ],
        compiler_params=pltpu.CompilerParams(use_tc_tiling_on_sc=False),
    )
    def sc_gather(table_hbm, idx_hbm, out_hbm, idx_v, rows_v, sem_r):
        wid = lax.axis_index("s") * info.num_cores + lax.axis_index("c")
        base = wid * bpw
        pltpu.sync_copy(idx_hbm.at[pl.ds(base, bpw)], idx_v)
        pltpu.async_copy(table_hbm.at[idx_v], rows_v, sem_r).wait()
        pltpu.sync_copy(rows_v, out_hbm.at[pl.ds(base, bpw)])

    return sc_gather


def _tc_body(g_ref, w_ref, b_ref, f_ref, o_ref):
    proj = lax.dot_general(
        g_ref[...], w_ref[...], (((1,), (1,)), ((), ())),
        preferred_element_type=jnp.float32,
    )
    proj = proj + b_ref[...]
    o_ref[...] = f_ref[...] + proj[:, None, :]


def _tc_proj_add(gathered, W, b2, features):
    grid = (_BATCH // _BB,)
    return pl.pallas_call(
        _tc_body,
        grid=grid,
        in_specs=[
            pl.BlockSpec((_BB, _EMBED_DIM), lambda i: (i, 0)),
            pl.BlockSpec((_FEATURE_DIM, _EMBED_DIM), lambda i: (0, 0)),
            pl.BlockSpec((1, _FEATURE_DIM), lambda i: (0, 0)),
            pl.BlockSpec((_BB, _SEQ_LEN, _FEATURE_DIM), lambda i: (i, 0, 0)),
        ],
        out_specs=pl.BlockSpec((_BB, _SEQ_LEN, _FEATURE_DIM), lambda i: (i, 0, 0)),
        out_shape=jax.ShapeDtypeStruct((_BATCH, _SEQ_LEN, _FEATURE_DIM), jnp.float32),
    )(gathered, W, b2, features)


def kernel(features, subject_ids, emb_table, W, b):
    ids = subject_ids.astype(jnp.int32)
    gathered = _build_sc_gather(_BATCH, _EMBED_DIM)(emb_table, ids)
    return _tc_proj_add(gathered, W, b.reshape(1, _FEATURE_DIM), features)


# speedup vs baseline: 1.0263x; 1.0263x over previous
import functools

import jax
import jax.numpy as jnp
from jax import lax
from jax.experimental import pallas as pl
from jax.experimental.pallas import tpu as pltpu
from jax.experimental.pallas import tpu_sc as plsc

_NUM_SUBJECTS = 100000
_EMBED_DIM = 64
_FEATURE_DIM = 128
_BATCH = 4096
_SEQ_LEN = 50

_CB = 32
_K = 8


@functools.cache
def _build_sc_gather(batch, embed_dim):
    info = plsc.get_sparse_core_info()
    nw = info.num_cores * info.num_subcores
    bpw = batch // nw
    mesh = plsc.VectorSubcoreMesh(core_axis_name="c", subcore_axis_name="s")

    @functools.partial(
        pl.kernel,
        mesh=mesh,
        out_type=jax.ShapeDtypeStruct((batch, embed_dim), jnp.float32),
        scratch_types=[
            pltpu.VMEM((bpw,), jnp.int32),
            pltpu.VMEM((bpw, embed_dim), jnp.float32),
            pltpu.SemaphoreType.DMA,
        ],
        compiler_params=pltpu.CompilerParams(use_tc_tiling_on_sc=False),
    )
    def sc_gather(table_hbm, idx_hbm, out_hbm, idx_v, rows_v, sem_r):
        wid = lax.axis_index("s") * info.num_cores + lax.axis_index("c")
        base = wid * bpw
        pltpu.sync_copy(idx_hbm.at[pl.ds(base, bpw)], idx_v)
        pltpu.async_copy(table_hbm.at[idx_v], rows_v, sem_r).wait()
        pltpu.sync_copy(rows_v, out_hbm.at[pl.ds(base, bpw)])

    return sc_gather


def _tc_body(g_ref, w_ref, b_ref, f_hbm, o_hbm, proj_ref, ibufs, obufs,
             in_sems, out_sems):
    proj_ref[...] = lax.dot_general(
        g_ref[...], w_ref[...], (((1,), (1,)), ((), ())),
        preferred_element_type=jnp.float32,
    ) + b_ref[...]

    n_groups = _BATCH // (_CB * _K)

    def in_copy(g, k):
        return pltpu.make_async_copy(
            f_hbm.at[pl.ds((g * _K + k) * _CB, _CB)], ibufs[k], in_sems[k])

    def out_copy(g, k):
        return pltpu.make_async_copy(
            obufs[k], o_hbm.at[pl.ds((g * _K + k) * _CB, _CB)], out_sems[k])

    for k in range(_K):
        in_copy(0, k).start()

    def group(g, _):
        for k in range(_K):
            @pl.when(g > 0)
            def _():
                out_copy(g - 1, k).wait()

            in_copy(g, k).wait()
            row0 = (g * _K + k) * _CB
            proj = proj_ref[pl.ds(row0, _CB), :]
            obufs[k][...] = ibufs[k][...] + proj[:, None, :]
            out_copy(g, k).start()

            @pl.when(g + 1 < n_groups)
            def _():
                in_copy(g + 1, k).start()
        return 0

    lax.fori_loop(0, n_groups, group, 0, unroll=False)
    for k in range(_K):
        out_copy(n_groups - 1, k).wait()


def _tc_proj_add(gathered, W, b2, features):
    scratch = [pltpu.VMEM((_BATCH, _FEATURE_DIM), jnp.float32)]
    scratch.append([pltpu.VMEM((_CB, _SEQ_LEN, _FEATURE_DIM), jnp.float32)
                    for _ in range(_K)])
    scratch.append([pltpu.VMEM((_CB, _SEQ_LEN, _FEATURE_DIM), jnp.float32)
                    for _ in range(_K)])
    scratch.append([pltpu.SemaphoreType.DMA for _ in range(_K)])
    scratch.append([pltpu.SemaphoreType.DMA for _ in range(_K)])
    return pl.pallas_call(
        _tc_body,
        in_specs=[
            pl.BlockSpec(memory_space=pltpu.VMEM),
            pl.BlockSpec(memory_space=pltpu.VMEM),
            pl.BlockSpec(memory_space=pltpu.VMEM),
            pl.BlockSpec(memory_space=pl.ANY),
        ],
        out_specs=pl.BlockSpec(memory_space=pl.ANY),
        out_shape=jax.ShapeDtypeStruct((_BATCH, _SEQ_LEN, _FEATURE_DIM), jnp.float32),
        scratch_shapes=scratch,
    )(gathered, W, b2, features)


def kernel(features, subject_ids, emb_table, W, b):
    ids = subject_ids.astype(jnp.int32)
    gathered = _build_sc_gather(_BATCH, _EMBED_DIM)(emb_table, ids)
    return _tc_proj_add(gathered, W, b.reshape(1, _FEATURE_DIM), features)


# speedup vs baseline: 1.0290x; 1.0026x over previous
import functools

import jax
import jax.numpy as jnp
from jax import lax
from jax.experimental import pallas as pl
from jax.experimental.pallas import tpu as pltpu
from jax.experimental.pallas import tpu_sc as plsc

_NUM_SUBJECTS = 100000
_EMBED_DIM = 64
_FEATURE_DIM = 128
_BATCH = 4096
_SEQ_LEN = 50

_CB = 32
_K = 8


@functools.cache
def _build_sc_gather(batch, embed_dim):
    info = plsc.get_sparse_core_info()
    nw = info.num_cores * info.num_subcores
    bpw = batch // nw
    mesh = plsc.VectorSubcoreMesh(core_axis_name="c", subcore_axis_name="s")

    @functools.partial(
        pl.kernel,
        mesh=mesh,
        out_type=jax.ShapeDtypeStruct((batch, embed_dim), jnp.float32),
        scratch_types=[
            pltpu.VMEM((bpw,), jnp.int32),
            pltpu.VMEM((bpw, embed_dim), jnp.float32),
            pltpu.SemaphoreType.DMA,
        ],
        compiler_params=pltpu.CompilerParams(use_tc_tiling_on_sc=False),
    )
    def sc_gather(table_hbm, idx_hbm, out_hbm, idx_v, rows_v, sem_r):
        wid = lax.axis_index("s") * info.num_cores + lax.axis_index("c")
        base = wid * bpw
        pltpu.sync_copy(idx_hbm.at[pl.ds(base, bpw)], idx_v)
        pltpu.async_copy(table_hbm.at[idx_v], rows_v, sem_r).wait()
        pltpu.sync_copy(rows_v, out_hbm.at[pl.ds(base, bpw)])

    return sc_gather


def _tc_body(g_ref, w_ref, b_ref, f_hbm, o_hbm, proj_ref, ibufs, obufs,
             in_sems, out_sems):
    proj_ref[...] = lax.dot_general(
        g_ref[...], w_ref[...], (((1,), (1,)), ((), ())),
        preferred_element_type=jnp.float32,
    ) + b_ref[...]

    n_groups = _BATCH // (_CB * _K)

    def in_copy(g, k):
        return pltpu.make_async_copy(
            f_hbm.at[pl.ds((g * _K + k) * _CB, _CB)], ibufs[k], in_sems[k])

    def out_copy(g, k):
        return pltpu.make_async_copy(
            obufs[k], o_hbm.at[pl.ds((g * _K + k) * _CB, _CB)], out_sems[k])

    for k in range(_K):
        in_copy(0, k).start(priority=k % 2)

    def group(g, _):
        for k in range(_K):
            @pl.when(g > 0)
            def _():
                out_copy(g - 1, k).wait()

            in_copy(g, k).wait()
            row0 = (g * _K + k) * _CB
            proj = proj_ref[pl.ds(row0, _CB), :]
            obufs[k][...] = ibufs[k][...] + proj[:, None, :]
            out_copy(g, k).start(priority=k % 2)

            @pl.when(g + 1 < n_groups)
            def _():
                in_copy(g + 1, k).start(priority=k % 2)
        return 0

    lax.fori_loop(0, n_groups, group, 0, unroll=False)
    for k in range(_K):
        out_copy(n_groups - 1, k).wait()


def _tc_proj_add(gathered, W, b2, features):
    scratch = [pltpu.VMEM((_BATCH, _FEATURE_DIM), jnp.float32)]
    scratch.append([pltpu.VMEM((_CB, _SEQ_LEN, _FEATURE_DIM), jnp.float32)
                    for _ in range(_K)])
    scratch.append([pltpu.VMEM((_CB, _SEQ_LEN, _FEATURE_DIM), jnp.float32)
                    for _ in range(_K)])
    scratch.append([pltpu.SemaphoreType.DMA for _ in range(_K)])
    scratch.append([pltpu.SemaphoreType.DMA for _ in range(_K)])
    return pl.pallas_call(
        _tc_body,
        in_specs=[
            pl.BlockSpec(memory_space=pltpu.VMEM),
            pl.BlockSpec(memory_space=pltpu.VMEM),
            pl.BlockSpec(memory_space=pltpu.VMEM),
            pl.BlockSpec(memory_space=pl.ANY),
        ],
        out_specs=pl.BlockSpec(memory_space=pl.ANY),
        out_shape=jax.ShapeDtypeStruct((_BATCH, _SEQ_LEN, _FEATURE_DIM), jnp.float32),
        scratch_shapes=scratch,
    )(gathered, W, b2, features)


def kernel(features, subject_ids, emb_table, W, b):
    ids = subject_ids.astype(jnp.int32)
    gathered = _build_sc_gather(_BATCH, _EMBED_DIM)(emb_table, ids)
    return _tc_proj_add(gathered, W, b.reshape(1, _FEATURE_DIM), features)


# speedup vs baseline: 1.1460x; 1.1136x over previous
import jax
import jax.numpy as jnp
from jax import lax
from jax.experimental import pallas as pl
from jax.experimental.pallas import tpu as pltpu

_NUM_SUBJECTS = 100000
_EMBED_DIM = 64
_FEATURE_DIM = 128
_BATCH = 4096
_SEQ_LEN = 50

_CB = 32
_K = 8
_GU = 8


def _tc_body(ids_ref, w_ref, b_ref, table_hbm, f_hbm, o_hbm,
             erows, proj_ref, ibufs, obufs, gsem, in_sems, out_sems):
    n_groups = _BATCH // (_CB * _K)

    def in_copy(g, k):
        return pltpu.make_async_copy(
            f_hbm.at[pl.ds((g * _K + k) * _CB, _CB)], ibufs[k], in_sems[k])

    def out_copy(g, k):
        return pltpu.make_async_copy(
            obufs[k], o_hbm.at[pl.ds((g * _K + k) * _CB, _CB)], out_sems[k])

    for k in range(_K):
        in_copy(0, k).start()

    def gissue(i, _):
        for j in range(_GU):
            r = i * _GU + j
            rid = ids_ref[r]
            pltpu.make_async_copy(
                table_hbm.at[pl.ds(rid, 1)], erows.at[pl.ds(r, 1)], gsem
            ).start()
        return 0

    lax.fori_loop(0, _BATCH // _GU, gissue, 0, unroll=False)

    def gwait(i, _):
        pltpu.make_async_copy(
            table_hbm.at[pl.ds(0, 64)], erows.at[pl.ds(i * 64, 64)], gsem
        ).wait()
        return 0

    lax.fori_loop(0, _BATCH // 64, gwait, 0, unroll=False)

    proj_ref[...] = lax.dot_general(
        erows[...], w_ref[...], (((1,), (1,)), ((), ())),
        preferred_element_type=jnp.float32,
    ) + b_ref[...]

    def group(g, _):
        for k in range(_K):
            @pl.when(g > 0)
            def _():
                out_copy(g - 1, k).wait()

            in_copy(g, k).wait()
            row0 = (g * _K + k) * _CB
            proj = proj_ref[pl.ds(row0, _CB), :]
            obufs[k][...] = ibufs[k][...] + proj[:, None, :]
            out_copy(g, k).start()

            @pl.when(g + 1 < n_groups)
            def _():
                in_copy(g + 1, k).start()
        return 0

    lax.fori_loop(0, n_groups, group, 0, unroll=False)
    for k in range(_K):
        out_copy(n_groups - 1, k).wait()


def kernel(features, subject_ids, emb_table, W, b):
    ids = subject_ids.astype(jnp.int32)
    scratch = [
        pltpu.VMEM((_BATCH, _EMBED_DIM), jnp.float32),
        pltpu.VMEM((_BATCH, _FEATURE_DIM), jnp.float32),
        [pltpu.VMEM((_CB, _SEQ_LEN, _FEATURE_DIM), jnp.float32)
         for _ in range(_K)],
        [pltpu.VMEM((_CB, _SEQ_LEN, _FEATURE_DIM), jnp.float32)
         for _ in range(_K)],
        pltpu.SemaphoreType.DMA,
        [pltpu.SemaphoreType.DMA for _ in range(_K)],
        [pltpu.SemaphoreType.DMA for _ in range(_K)],
    ]
    return pl.pallas_call(
        _tc_body,
        in_specs=[
            pl.BlockSpec(memory_space=pltpu.SMEM),
            pl.BlockSpec(memory_space=pltpu.VMEM),
            pl.BlockSpec(memory_space=pltpu.VMEM),
            pl.BlockSpec(memory_space=pl.ANY),
            pl.BlockSpec(memory_space=pl.ANY),
        ],
        out_specs=pl.BlockSpec(memory_space=pl.ANY),
        out_shape=jax.ShapeDtypeStruct((_BATCH, _SEQ_LEN, _FEATURE_DIM), jnp.float32),
        scratch_shapes=scratch,
    )(ids, W, b.reshape(1, _FEATURE_DIM), emb_table, features)


# speedup vs baseline: 2.5004x; 2.1820x over previous
import jax
import jax.numpy as jnp
from jax import lax
from jax.experimental import pallas as pl
from jax.experimental.pallas import tpu as pltpu

_NUM_SUBJECTS = 100000
_EMBED_DIM = 64
_FEATURE_DIM = 128
_BATCH = 4096
_SEQ_LEN = 50

_K = 5
_GU = 8


def _tc_body(ids_ref, w_ref, b_ref, table_hbm, f_hbm, o_hbm,
             erows, proj_ref, ibufs, obufs, gsem, in_sems, out_sems):
    n_groups = _SEQ_LEN // _K

    def in_copy(g, k):
        return pltpu.make_async_copy(
            f_hbm.at[pl.ds(g * _K + k, 1)], ibufs[k], in_sems[k])

    def out_copy(g, k):
        return pltpu.make_async_copy(
            obufs[k], o_hbm.at[pl.ds(g * _K + k, 1)], out_sems[k])

    for k in range(_K):
        in_copy(0, k).start()

    def gissue(i, _):
        for j in range(_GU):
            r = i * _GU + j
            rid = ids_ref[r]
            pltpu.make_async_copy(
                table_hbm.at[pl.ds(rid, 1)], erows.at[pl.ds(r, 1)], gsem
            ).start()
        return 0

    lax.fori_loop(0, _BATCH // _GU, gissue, 0, unroll=False)

    def gwait(i, _):
        pltpu.make_async_copy(
            table_hbm.at[pl.ds(0, 64)], erows.at[pl.ds(i * 64, 64)], gsem
        ).wait()
        return 0

    lax.fori_loop(0, _BATCH // 64, gwait, 0, unroll=False)

    proj_ref[...] = lax.dot_general(
        erows[...], w_ref[...], (((1,), (0,)), ((), ())),
        preferred_element_type=jnp.float32,
    ) + b_ref[...]

    def group(g, _):
        for k in range(_K):
            @pl.when(g > 0)
            def _():
                out_copy(g - 1, k).wait()

            in_copy(g, k).wait()
            obufs[k][...] = ibufs[k][...] + proj_ref[...][None, :, :]
            out_copy(g, k).start()

            @pl.when(g + 1 < n_groups)
            def _():
                in_copy(g + 1, k).start()
        return 0

    lax.fori_loop(0, n_groups, group, 0, unroll=False)
    for k in range(_K):
        out_copy(n_groups - 1, k).wait()


def kernel(features, subject_ids, emb_table, W, b):
    ids = subject_ids.astype(jnp.int32)
    ft = jnp.transpose(features, (1, 0, 2))
    wt = jnp.transpose(W, (1, 0))
    scratch = [
        pltpu.VMEM((_BATCH, _EMBED_DIM), jnp.float32),
        pltpu.VMEM((_BATCH, _FEATURE_DIM), jnp.float32),
        [pltpu.VMEM((1, _BATCH, _FEATURE_DIM), jnp.float32)
         for _ in range(_K)],
        [pltpu.VMEM((1, _BATCH, _FEATURE_DIM), jnp.float32)
         for _ in range(_K)],
        pltpu.SemaphoreType.DMA,
        [pltpu.SemaphoreType.DMA for _ in range(_K)],
        [pltpu.SemaphoreType.DMA for _ in range(_K)],
    ]
    out_t = pl.pallas_call(
        _tc_body,
        in_specs=[
            pl.BlockSpec(memory_space=pltpu.SMEM),
            pl.BlockSpec(memory_space=pltpu.VMEM),
            pl.BlockSpec(memory_space=pltpu.VMEM),
            pl.BlockSpec(memory_space=pl.ANY),
            pl.BlockSpec(memory_space=pl.ANY),
        ],
        out_specs=pl.BlockSpec(memory_space=pl.ANY),
        out_shape=jax.ShapeDtypeStruct((_SEQ_LEN, _BATCH, _FEATURE_DIM), jnp.float32),
        scratch_shapes=scratch,
    )(ids, wt, b.reshape(1, _FEATURE_DIM), emb_table, ft)
    return jnp.transpose(out_t, (1, 0, 2))


# speedup vs baseline: 2.8073x; 1.1227x over previous
import jax
import jax.numpy as jnp
from jax import lax
from jax.experimental import pallas as pl
from jax.experimental.pallas import tpu as pltpu

_NUM_SUBJECTS = 100000
_EMBED_DIM = 64
_FEATURE_DIM = 128
_BATCH = 4096
_SEQ_LEN = 50

_CB = 128
_K = 4
_NC = _BATCH // _CB
_GU = 8


def _tc_body(ids_ref, w_ref, b_ref, table_hbm, f_hbm, o_hbm,
             erows, ibufs, obufs, gsem, in_sems, out_sems):
    n_groups = _NC // _K

    def in_copy(c, k):
        return pltpu.make_async_copy(
            f_hbm.at[:, pl.ds(c * _CB, _CB), :], ibufs[k], in_sems[k])

    def out_copy(c, k):
        return pltpu.make_async_copy(
            obufs[k], o_hbm.at[:, pl.ds(c * _CB, _CB), :], out_sems[k])

    def gather_issue(c):
        def gissue(i, _):
            for j in range(_GU):
                r = c * _CB + i * _GU + j
                rid = ids_ref[r]
                pltpu.make_async_copy(
                    table_hbm.at[pl.ds(rid, 1)], erows.at[pl.ds(r, 1)], gsem
                ).start()
            return 0
        lax.fori_loop(0, _CB // _GU, gissue, 0, unroll=False)

    def gather_wait(c):
        pltpu.make_async_copy(
            table_hbm.at[pl.ds(0, _CB)], erows.at[pl.ds(c * _CB, _CB)], gsem
        ).wait()

    for k in range(_K):
        in_copy(k, k).start()
    for c in range(2 * _K):
        gather_issue(c)

    def group(g, _):
        for k in range(_K):
            c = g * _K + k

            @pl.when(g > 0)
            def _():
                out_copy(c - _K, k).wait()

            @pl.when(g + 2 < n_groups)
            def _():
                gather_issue(c + 2 * _K)

            gather_wait(c)
            proj = lax.dot_general(
                erows[pl.ds(c * _CB, _CB), :], w_ref[...],
                (((1,), (0,)), ((), ())),
                preferred_element_type=jnp.float32,
            ) + b_ref[...]
            in_copy(c, k).wait()
            obufs[k][...] = ibufs[k][...] + proj[None, :, :]
            out_copy(c, k).start()

            @pl.when(g + 1 < n_groups)
            def _():
                in_copy(c + _K, k).start()
        return 0

    lax.fori_loop(0, n_groups, group, 0, unroll=False)
    for k in range(_K):
        out_copy(_NC - _K + k, k).wait()


def kernel(features, subject_ids, emb_table, W, b):
    ids = subject_ids.astype(jnp.int32)
    ft = jnp.transpose(features, (1, 0, 2))
    wt = jnp.transpose(W, (1, 0))
    scratch = [
        pltpu.VMEM((_BATCH, _EMBED_DIM), jnp.float32),
        [pltpu.VMEM((_SEQ_LEN, _CB, _FEATURE_DIM), jnp.float32)
         for _ in range(_K)],
        [pltpu.VMEM((_SEQ_LEN, _CB, _FEATURE_DIM), jnp.float32)
         for _ in range(_K)],
        pltpu.SemaphoreType.DMA,
        [pltpu.SemaphoreType.DMA for _ in range(_K)],
        [pltpu.SemaphoreType.DMA for _ in range(_K)],
    ]
    out_t = pl.pallas_call(
        _tc_body,
        in_specs=[
            pl.BlockSpec(memory_space=pltpu.SMEM),
            pl.BlockSpec(memory_space=pltpu.VMEM),
            pl.BlockSpec(memory_space=pltpu.VMEM),
            pl.BlockSpec(memory_space=pl.ANY),
            pl.BlockSpec(memory_space=pl.ANY),
        ],
        out_specs=pl.BlockSpec(memory_space=pl.ANY),
        out_shape=jax.ShapeDtypeStruct((_SEQ_LEN, _BATCH, _FEATURE_DIM), jnp.float32),
        scratch_shapes=scratch,
    )(ids, wt, b.reshape(1, _FEATURE_DIM), emb_table, ft)
    return jnp.transpose(out_t, (1, 0, 2))


# speedup vs baseline: 2.8610x; 1.0191x over previous
import jax
import jax.numpy as jnp
from jax import lax
from jax.experimental import pallas as pl
from jax.experimental.pallas import tpu as pltpu

_NUM_SUBJECTS = 100000
_EMBED_DIM = 64
_FEATURE_DIM = 128
_BATCH = 4096
_SEQ_LEN = 50

_CB = 128
_K = 4
_NC = _BATCH // _CB
_GU = 8


def _tc_body(ids_ref, w_ref, b_ref, table_hbm, f_hbm, o_hbm,
             erows, ibufs, obufs, gsems, in_sems, out_sems):
    n_groups = _NC // _K

    def in_copy(c, k):
        return pltpu.make_async_copy(
            f_hbm.at[:, pl.ds(c * _CB, _CB), :], ibufs[k], in_sems[k])

    def out_copy(c, k):
        return pltpu.make_async_copy(
            obufs[k], o_hbm.at[:, pl.ds(c * _CB, _CB), :], out_sems[k])

    def gather_issue(c, k):
        def gissue(i, _):
            for j in range(_GU):
                r = c * _CB + i * _GU + j
                rid = ids_ref[r]
                pltpu.make_async_copy(
                    table_hbm.at[pl.ds(rid, 1)], erows.at[pl.ds(r, 1)],
                    gsems[k],
                ).start()
            return 0
        lax.fori_loop(0, _CB // _GU, gissue, 0, unroll=False)

    def gather_wait(c, k):
        pltpu.make_async_copy(
            table_hbm.at[pl.ds(0, _CB)], erows.at[pl.ds(c * _CB, _CB)],
            gsems[k],
        ).wait()

    for k in range(_K):
        in_copy(k, k).start()
        gather_issue(k, k)

    def group(g, _):
        for k in range(_K):
            c = g * _K + k

            @pl.when(g > 0)
            def _():
                out_copy(c - _K, k).wait()

            gather_wait(c, k)

            @pl.when(g + 1 < n_groups)
            def _():
                gather_issue(c + _K, k)

            proj = lax.dot_general(
                erows[pl.ds(c * _CB, _CB), :], w_ref[...],
                (((1,), (0,)), ((), ())),
                preferred_element_type=jnp.float32,
            ) + b_ref[...]
            in_copy(c, k).wait()
            obufs[k][...] = ibufs[k][...] + proj[None, :, :]
            out_copy(c, k).start()

            @pl.when(g + 1 < n_groups)
            def _():
                in_copy(c + _K, k).start()
        return 0

    lax.fori_loop(0, n_groups, group, 0, unroll=False)
    for k in range(_K):
        out_copy(_NC - _K + k, k).wait()


def kernel(features, subject_ids, emb_table, W, b):
    ids = subject_ids.astype(jnp.int32)
    ft = jnp.transpose(features, (1, 0, 2))
    wt = jnp.transpose(W, (1, 0))
    scratch = [
        pltpu.VMEM((_BATCH, _EMBED_DIM), jnp.float32),
        [pltpu.VMEM((_SEQ_LEN, _CB, _FEATURE_DIM), jnp.float32)
         for _ in range(_K)],
        [pltpu.VMEM((_SEQ_LEN, _CB, _FEATURE_DIM), jnp.float32)
         for _ in range(_K)],
        [pltpu.SemaphoreType.DMA for _ in range(_K)],
        [pltpu.SemaphoreType.DMA for _ in range(_K)],
        [pltpu.SemaphoreType.DMA for _ in range(_K)],
    ]
    out_t = pl.pallas_call(
        _tc_body,
        in_specs=[
            pl.BlockSpec(memory_space=pltpu.SMEM),
            pl.BlockSpec(memory_space=pltpu.VMEM),
            pl.BlockSpec(memory_space=pltpu.VMEM),
            pl.BlockSpec(memory_space=pl.ANY),
            pl.BlockSpec(memory_space=pl.ANY),
        ],
        out_specs=pl.BlockSpec(memory_space=pl.ANY),
        out_shape=jax.ShapeDtypeStruct((_SEQ_LEN, _BATCH, _FEATURE_DIM), jnp.float32),
        scratch_shapes=scratch,
    )(ids, wt, b.reshape(1, _FEATURE_DIM), emb_table, ft)
    return jnp.transpose(out_t, (1, 0, 2))
